# baseline (speedup 1.0000x reference)
import functools

import jax
import jax.numpy as jnp
from jax import lax
from jax.experimental import pallas as pl
from jax.experimental.pallas import tpu as pltpu
from jax.experimental.pallas import tpu_sc as plsc

N = 10000
E = 320000
FEAT = 128
DIM = 32
NUM_GRAPHS = 64
NUM_CLASSES = 10

NC = 2
NS = 16
NW = NC * NS
CH = 128
SPB = 5
BLK = CH * SPB
NBLK = 16
EPW = BLK * NBLK
EP = NW * EPW
NCHK = EPW // CH
GPB = BLK // 16
RPT = N // NS


def _sc_aggregate(r, srcm, dstm, wm, zer):
    mesh = plsc.VectorSubcoreMesh(
        core_axis_name="c", subcore_axis_name="s", num_cores=NC, num_subcores=NS)

    @functools.partial(
        pl.kernel,
        out_type=jax.ShapeDtypeStruct((NC, NS, RPT, DIM), jnp.float32),
        mesh=mesh,
        scratch_types=[
            pltpu.VMEM((NCHK, CH), jnp.int32),
            pltpu.VMEM((NCHK, CH), jnp.int32),
            pltpu.VMEM((NCHK, CH), jnp.float32),
            pltpu.VMEM((BLK, DIM), jnp.float32),
            pltpu.VMEM((BLK, DIM), jnp.float32),
            pltpu.VMEM((BLK, DIM), jnp.float32),
            pltpu.VMEM_SHARED((N, DIM), jnp.float32),
            pltpu.SemaphoreType.DMA,
            pltpu.SemaphoreType.DMA,
            pltpu.SemaphoreType.DMA,
        ],
        compiler_params=pltpu.CompilerParams(use_tc_tiling_on_sc=False),
    )
    def agg(r_hbm, src_hbm, dst_hbm, w_hbm, z_hbm, out_hbm,
            src_v, dst_v, w_v, rows_a, rows_b, rows_c, acc_sh, gsem, ssem, zsem):
        c = lax.axis_index("c")
        s = lax.axis_index("s")
        wid = s * NC + c
        rows = (rows_a, rows_b, rows_c)

        zd = pltpu.async_copy(z_hbm.at[s], acc_sh.at[pl.ds(s * RPT, RPT)], zsem)

        pltpu.sync_copy(src_hbm.at[wid], src_v)
        pltpu.sync_copy(dst_hbm.at[wid], dst_v)
        pltpu.sync_copy(w_hbm.at[wid], w_v)

        def fire_gathers(b):
            buf = rows[b % 3]
            return [pltpu.async_copy(r_hbm.at[src_v.at[b * SPB + j]],
                                     buf.at[pl.ds(j * CH, CH)], gsem)
                    for j in range(SPB)]

        def fire_scatters(b):
            buf = rows[b % 3]
            return [pltpu.async_copy(buf.at[pl.ds(j * CH, CH)],
                                     acc_sh.at[dst_v.at[b * SPB + j]],
                                     ssem, add=True)
                    for j in range(SPB)]

        gds = [None] * NBLK
        sds = [None] * NBLK
        gds[0] = fire_gathers(0)
        for b in range(NBLK):
            buf = rows[b % 3]
            if b + 1 < NBLK:
                if b >= 2:
                    for d_ in sds[b - 2]:
                        d_.wait()
                gds[b + 1] = fire_gathers(b + 1)
            for d_ in gds[b]:
                d_.wait()
            if b == 0:
                zd.wait()
                plsc.subcore_barrier()

            def mul_body(g, carry, b=b, buf=buf):
                gg = b * GPB + g
                jrow = gg // (CH // 16)
                jcol = (gg % (CH // 16)) * 16
                wrow = w_v[jrow, pl.ds(jcol, 16)]
                for k in range(16):
                    e = g * 16 + k
                    wv = jnp.broadcast_to(wrow[k], (16,))
                    buf[e, pl.ds(0, 16)] = buf[e, pl.ds(0, 16)] * wv
                    buf[e, pl.ds(16, 16)] = buf[e, pl.ds(16, 16)] * wv
                return carry
            lax.fori_loop(0, GPB, mul_body, 0)
            sds[b] = fire_scatters(b)

        for b in (NBLK - 3, NBLK - 2, NBLK - 1):
            for d_ in sds[b]:
                d_.wait()
        plsc.subcore_barrier()
        pltpu.sync_copy(acc_sh.at[pl.ds(s * RPT, RPT)], out_hbm.at[c, s])

    return agg(r, srcm, dstm, wm, zer)


def _tc_first(x, Wc):
    def body(x_ref, w_ref, r_ref, root_ref):
        y = jnp.dot(x_ref[...], w_ref[...].T, preferred_element_type=jnp.float32)
        r_ref[...] = y[:, :DIM]
        root_ref[...] = y[:, DIM:]
    return pl.pallas_call(
        body,
        out_shape=(jax.ShapeDtypeStruct((N, DIM), jnp.float32),
                   jax.ShapeDtypeStruct((N, DIM), jnp.float32)),
    )(x, Wc)


def _tc_mid(aggp, b, root, Wc):
    def body(a_ref, b_ref, rt_ref, w_ref, r_ref, root_ref):
        h = jnp.maximum(a_ref[0] + a_ref[1] + b_ref[...] + rt_ref[...], 0.0)
        y = jnp.dot(h, w_ref[...].T, preferred_element_type=jnp.float32)
        r_ref[...] = y[:, :DIM]
        root_ref[...] = y[:, DIM:]
    return pl.pallas_call(
        body,
        out_shape=(jax.ShapeDtypeStruct((N, DIM), jnp.float32),
                   jax.ShapeDtypeStruct((N, DIM), jnp.float32)),
    )(aggp, b, root, Wc)


def _tc_final(aggp, b5, root5, batch2d, Wl1, bl1, Wl2, bl2):
    def body(a_ref, b_ref, rt_ref, bt_ref, w1_ref, c1_ref, w2_ref, c2_ref, o_ref):
        h = jnp.maximum(a_ref[0] + a_ref[1] + b_ref[...] + rt_ref[...], 0.0)
        ids = bt_ref[...]
        oh = (lax.broadcasted_iota(jnp.int32, (NUM_GRAPHS, N), 0) == ids
              ).astype(jnp.float32)
        pooled = jnp.dot(oh, h, preferred_element_type=jnp.float32)
        t = jnp.maximum(
            jnp.dot(pooled, w1_ref[...].T, preferred_element_type=jnp.float32)
            + c1_ref[...], 0.0)
        logits = (jnp.dot(t, w2_ref[...].T, preferred_element_type=jnp.float32)
                  + c2_ref[...])
        m = jnp.max(logits, axis=-1, keepdims=True)
        sh = logits - m
        o_ref[...] = sh - jnp.log(jnp.sum(jnp.exp(sh), axis=-1, keepdims=True))
    return pl.pallas_call(
        body,
        out_shape=jax.ShapeDtypeStruct((NUM_GRAPHS, NUM_CLASSES), jnp.float32),
    )(aggp, b5, root5, batch2d, Wl1, bl1, Wl2, bl2)


def kernel(x, edge_index, batch, edge_weight,
           W1_rel, b1, W1_root, W2_rel, b2, W2_root, W3_rel, b3, W3_root,
           W4_rel, b4, W4_root, W5_rel, b5, W5_root, Wl1, bl1, Wl2, bl2):
    pad_i = jnp.zeros((EP - E,), jnp.int32)
    srcm = jnp.concatenate([edge_index[0], pad_i]).reshape(NW, NCHK, CH)
    dstm = jnp.concatenate([edge_index[1], pad_i]).reshape(NW, NCHK, CH)
    wm = jnp.concatenate([edge_weight, jnp.zeros((EP - E,), jnp.float32)]
                         ).reshape(NW, NCHK, CH)
    zer = jnp.zeros((NS, RPT, DIM), jnp.float32)

    r, root = _tc_first(x, jnp.concatenate([W1_rel, W1_root], axis=0))
    for b_prev, Wr, Wt in ((b1, W2_rel, W2_root), (b2, W3_rel, W3_root),
                           (b3, W4_rel, W4_root), (b4, W5_rel, W5_root)):
        aggp = _sc_aggregate(r, srcm, dstm, wm, zer).reshape(NC, N, DIM)
        r, root = _tc_mid(aggp, b_prev.reshape(1, DIM), root,
                          jnp.concatenate([Wr, Wt], axis=0))
    aggp = _sc_aggregate(r, srcm, dstm, wm, zer).reshape(NC, N, DIM)
    return _tc_final(aggp, b5.reshape(1, DIM), root, batch.reshape(1, N),
                     Wl1, bl1.reshape(1, DIM), Wl2, bl2.reshape(1, NUM_CLASSES))


# speedup vs baseline: 2.2039x; 2.2039x over previous
import functools

import jax
import jax.numpy as jnp
from jax import lax
from jax.experimental import pallas as pl
from jax.experimental.pallas import tpu as pltpu
from jax.experimental.pallas import tpu_sc as plsc

N = 10000
E = 320000
FEAT = 128
DIM = 32
NUM_GRAPHS = 64
NUM_CLASSES = 10

NC = 2
NS = 16
NW = NC * NS
CH = 80
SPB = 5
BLK = CH * SPB
NBLK = 25
EPW = BLK * NBLK
EP = NW * EPW
NCHK = EPW // CH
GPB = BLK // 16
RPT = N // NS


def _sc_aggregate(r, srcm, dstm, wm, zer):
    mesh = plsc.VectorSubcoreMesh(
        core_axis_name="c", subcore_axis_name="s", num_cores=NC, num_subcores=NS)

    @functools.partial(
        pl.kernel,
        out_type=jax.ShapeDtypeStruct((NC, NS, RPT, DIM), jnp.float32),
        mesh=mesh,
        scratch_types=[
            pltpu.VMEM((NCHK, CH), jnp.int32),
            pltpu.VMEM((NCHK, CH), jnp.int32),
            pltpu.VMEM((NCHK, CH), jnp.float32),
            pltpu.VMEM((BLK, DIM), jnp.float32),
            pltpu.VMEM((BLK, DIM), jnp.float32),
            pltpu.VMEM((BLK, DIM), jnp.float32),
            pltpu.VMEM_SHARED((N, DIM), jnp.float32),
            pltpu.SemaphoreType.DMA,
            pltpu.SemaphoreType.DMA,
            pltpu.SemaphoreType.DMA,
        ],
        compiler_params=pltpu.CompilerParams(use_tc_tiling_on_sc=False),
    )
    def agg(r_hbm, src_hbm, dst_hbm, w_hbm, z_hbm, out_hbm,
            src_v, dst_v, w_v, rows_a, rows_b, rows_c, acc_sh, gsem, ssem, zsem):
        c = lax.axis_index("c")
        s = lax.axis_index("s")
        wid = s * NC + c
        rows = (rows_a, rows_b, rows_c)

        zd = pltpu.async_copy(z_hbm.at[s], acc_sh.at[pl.ds(s * RPT, RPT)], zsem)

        pltpu.sync_copy(src_hbm.at[wid], src_v)
        pltpu.sync_copy(dst_hbm.at[wid], dst_v)
        pltpu.sync_copy(w_hbm.at[wid], w_v)

        def fire_gathers(b):
            buf = rows[b % 3]
            return [pltpu.async_copy(r_hbm.at[src_v.at[b * SPB + j]],
                                     buf.at[pl.ds(j * CH, CH)], gsem)
                    for j in range(SPB)]

        def fire_scatters(b):
            buf = rows[b % 3]
            return [pltpu.async_copy(buf.at[pl.ds(j * CH, CH)],
                                     acc_sh.at[dst_v.at[b * SPB + j]],
                                     ssem, add=True)
                    for j in range(SPB)]

        gds = [None] * NBLK
        sds = [None] * NBLK
        gds[0] = fire_gathers(0)
        for b in range(NBLK):
            buf = rows[b % 3]
            if b + 1 < NBLK:
                if b >= 2:
                    for d_ in sds[b - 2]:
                        d_.wait()
                gds[b + 1] = fire_gathers(b + 1)
            for d_ in gds[b]:
                d_.wait()
            if b == 0:
                zd.wait()
                plsc.subcore_barrier()

            def mul_body(g, carry, b=b, buf=buf):
                gg = b * GPB + g
                jrow = gg // (CH // 16)
                jcol = (gg % (CH // 16)) * 16
                wrow = w_v[jrow, pl.ds(jcol, 16)]
                for k in range(16):
                    e = g * 16 + k
                    wv = jnp.broadcast_to(wrow[k], (16,))
                    buf[e, pl.ds(0, 16)] = buf[e, pl.ds(0, 16)] * wv
                    buf[e, pl.ds(16, 16)] = buf[e, pl.ds(16, 16)] * wv
                return carry
            lax.fori_loop(0, GPB, mul_body, 0)
            sds[b] = fire_scatters(b)

        for b in (NBLK - 3, NBLK - 2, NBLK - 1):
            for d_ in sds[b]:
                d_.wait()
        plsc.subcore_barrier()
        pltpu.sync_copy(acc_sh.at[pl.ds(s * RPT, RPT)], out_hbm.at[c, s])

    return agg(r, srcm, dstm, wm, zer)


def _tc_first(x, Wc):
    def body(x_ref, w_ref, r_ref, root_ref):
        y = jnp.dot(x_ref[...], w_ref[...].T, preferred_element_type=jnp.float32)
        r_ref[...] = y[:, :DIM]
        root_ref[...] = y[:, DIM:]
    return pl.pallas_call(
        body,
        out_shape=(jax.ShapeDtypeStruct((N, DIM), jnp.float32),
                   jax.ShapeDtypeStruct((N, DIM), jnp.float32)),
    )(x, Wc)


def _tc_mid(aggp, b, root, Wc):
    def body(a_ref, b_ref, rt_ref, w_ref, r_ref, root_ref):
        h = jnp.maximum(a_ref[0] + a_ref[1] + b_ref[...] + rt_ref[...], 0.0)
        y = jnp.dot(h, w_ref[...].T, preferred_element_type=jnp.float32)
        r_ref[...] = y[:, :DIM]
        root_ref[...] = y[:, DIM:]
    return pl.pallas_call(
        body,
        out_shape=(jax.ShapeDtypeStruct((N, DIM), jnp.float32),
                   jax.ShapeDtypeStruct((N, DIM), jnp.float32)),
    )(aggp, b, root, Wc)


def _tc_final(aggp, b5, root5, batch2d, Wl1, bl1, Wl2, bl2):
    def body(a_ref, b_ref, rt_ref, bt_ref, w1_ref, c1_ref, w2_ref, c2_ref, o_ref):
        h = jnp.maximum(a_ref[0] + a_ref[1] + b_ref[...] + rt_ref[...], 0.0)
        ids = bt_ref[...]
        oh = (lax.broadcasted_iota(jnp.int32, (NUM_GRAPHS, N), 0) == ids
              ).astype(jnp.float32)
        pooled = jnp.dot(oh, h, preferred_element_type=jnp.float32)
        t = jnp.maximum(
            jnp.dot(pooled, w1_ref[...].T, preferred_element_type=jnp.float32)
            + c1_ref[...], 0.0)
        logits = (jnp.dot(t, w2_ref[...].T, preferred_element_type=jnp.float32)
                  + c2_ref[...])
        m = jnp.max(logits, axis=-1, keepdims=True)
        sh = logits - m
        o_ref[...] = sh - jnp.log(jnp.sum(jnp.exp(sh), axis=-1, keepdims=True))
    return pl.pallas_call(
        body,
        out_shape=jax.ShapeDtypeStruct((NUM_GRAPHS, NUM_CLASSES), jnp.float32),
    )(aggp, b5, root5, batch2d, Wl1, bl1, Wl2, bl2)


def kernel(x, edge_index, batch, edge_weight,
           W1_rel, b1, W1_root, W2_rel, b2, W2_root, W3_rel, b3, W3_root,
           W4_rel, b4, W4_root, W5_rel, b5, W5_root, Wl1, bl1, Wl2, bl2):
    srcm = edge_index[0].reshape(NW, NCHK, CH)
    dstm = edge_index[1].reshape(NW, NCHK, CH)
    wm = edge_weight.reshape(NW, NCHK, CH)
    zer = jnp.zeros((NS, RPT, DIM), jnp.float32)

    r, root = _tc_first(x, jnp.concatenate([W1_rel, W1_root], axis=0))
    for b_prev, Wr, Wt in ((b1, W2_rel, W2_root), (b2, W3_rel, W3_root),
                           (b3, W4_rel, W4_root), (b4, W5_rel, W5_root)):
        aggp = _sc_aggregate(r, srcm, dstm, wm, zer).reshape(NC, N, DIM)
        r, root = _tc_mid(aggp, b_prev.reshape(1, DIM), root,
                          jnp.concatenate([Wr, Wt], axis=0))
    aggp = _sc_aggregate(r, srcm, dstm, wm, zer).reshape(NC, N, DIM)
    return _tc_final(aggp, b5.reshape(1, DIM), root, batch.reshape(1, N),
                     Wl1, bl1.reshape(1, DIM), Wl2, bl2.reshape(1, NUM_CLASSES))


# speedup vs baseline: 2.2041x; 1.0001x over previous
import functools

import jax
import jax.numpy as jnp
from jax import lax
from jax.experimental import pallas as pl
from jax.experimental.pallas import tpu as pltpu
from jax.experimental.pallas import tpu_sc as plsc

N = 10000
E = 320000
FEAT = 128
DIM = 32
NUM_GRAPHS = 64
NUM_CLASSES = 10

NC = 2
NS = 16
NW = NC * NS
CH = 80
SPB = 5
BLK = CH * SPB
NBLK = 25
EPW = BLK * NBLK
EP = NW * EPW
NCHK = EPW // CH
GPB = BLK // 16
RPT = N // NS


def _sc_aggregate(r, srcm, dstm, wm, zer):
    mesh = plsc.VectorSubcoreMesh(
        core_axis_name="c", subcore_axis_name="s", num_cores=NC, num_subcores=NS)

    @functools.partial(
        pl.kernel,
        out_type=jax.ShapeDtypeStruct((NC, NS, RPT, DIM), jnp.float32),
        mesh=mesh,
        scratch_types=[
            pltpu.VMEM((NCHK, CH), jnp.int32),
            pltpu.VMEM((NCHK, CH), jnp.int32),
            pltpu.VMEM((EPW + 16,), jnp.float32),
            pltpu.VMEM((BLK, DIM), jnp.float32),
            pltpu.VMEM((BLK, DIM), jnp.float32),
            pltpu.VMEM((BLK, DIM), jnp.float32),
            pltpu.VMEM_SHARED((N, DIM), jnp.float32),
            pltpu.SemaphoreType.DMA,
            pltpu.SemaphoreType.DMA,
            pltpu.SemaphoreType.DMA,
        ],
        compiler_params=pltpu.CompilerParams(use_tc_tiling_on_sc=False),
    )
    def agg(r_hbm, src_hbm, dst_hbm, w_hbm, z_hbm, out_hbm,
            src_v, dst_v, w_v, rows_a, rows_b, rows_c,
            acc_sh, gsem, ssem, zsem):
        c = lax.axis_index("c")
        s = lax.axis_index("s")
        wid = s * NC + c
        rows = (rows_a, rows_b, rows_c)

        zd = pltpu.async_copy(z_hbm.at[s], acc_sh.at[pl.ds(s * RPT, RPT)], zsem)

        pltpu.sync_copy(src_hbm.at[wid], src_v)
        pltpu.sync_copy(dst_hbm.at[wid], dst_v)
        pltpu.sync_copy(w_hbm.at[wid, 0], w_v.at[pl.ds(0, EPW)])

        def fire_gathers(b):
            buf = rows[b % 3]
            return [pltpu.async_copy(r_hbm.at[src_v.at[b * SPB + j]],
                                     buf.at[pl.ds(j * CH, CH)], gsem)
                    for j in range(SPB)]

        def fire_scatters(b):
            buf = rows[b % 3]
            return [pltpu.async_copy(buf.at[pl.ds(j * CH, CH)],
                                     acc_sh.at[dst_v.at[b * SPB + j]],
                                     ssem, add=True)
                    for j in range(SPB)]

        gds = [None] * NBLK
        sds = [None] * NBLK
        gds[0] = fire_gathers(0)
        for b in range(NBLK):
            buf = rows[b % 3]
            if b + 1 < NBLK:
                if b >= 2:
                    for d_ in sds[b - 2]:
                        d_.wait()
                gds[b + 1] = fire_gathers(b + 1)
            for d_ in gds[b]:
                d_.wait()
            if b == 0:
                zd.wait()
                plsc.subcore_barrier()

            @plsc.parallel_loop(0, BLK, unroll=8)
            def _(e, b=b, buf=buf):
                wrow = w_v[pl.ds(b * BLK + e, 16)]
                wv = jnp.broadcast_to(wrow[0], (16,))
                buf[e, pl.ds(0, 16)] = buf[e, pl.ds(0, 16)] * wv
                buf[e, pl.ds(16, 16)] = buf[e, pl.ds(16, 16)] * wv
            sds[b] = fire_scatters(b)

        for b in (NBLK - 3, NBLK - 2, NBLK - 1):
            for d_ in sds[b]:
                d_.wait()
        plsc.subcore_barrier()
        pltpu.sync_copy(acc_sh.at[pl.ds(s * RPT, RPT)], out_hbm.at[c, s])

    return agg(r, srcm, dstm, wm, zer)


def _tc_first(x, Wc):
    def body(x_ref, w_ref, r_ref, root_ref):
        y = jnp.dot(x_ref[...], w_ref[...].T, preferred_element_type=jnp.float32)
        r_ref[...] = y[:, :DIM]
        root_ref[...] = y[:, DIM:]
    return pl.pallas_call(
        body,
        out_shape=(jax.ShapeDtypeStruct((N, DIM), jnp.float32),
                   jax.ShapeDtypeStruct((N, DIM), jnp.float32)),
    )(x, Wc)


def _tc_mid(aggp, b, root, Wc):
    def body(a_ref, b_ref, rt_ref, w_ref, r_ref, root_ref):
        h = jnp.maximum(a_ref[0] + a_ref[1] + b_ref[...] + rt_ref[...], 0.0)
        y = jnp.dot(h, w_ref[...].T, preferred_element_type=jnp.float32)
        r_ref[...] = y[:, :DIM]
        root_ref[...] = y[:, DIM:]
    return pl.pallas_call(
        body,
        out_shape=(jax.ShapeDtypeStruct((N, DIM), jnp.float32),
                   jax.ShapeDtypeStruct((N, DIM), jnp.float32)),
    )(aggp, b, root, Wc)


def _tc_final(aggp, b5, root5, batch2d, Wl1, bl1, Wl2, bl2):
    def body(a_ref, b_ref, rt_ref, bt_ref, w1_ref, c1_ref, w2_ref, c2_ref, o_ref):
        h = jnp.maximum(a_ref[0] + a_ref[1] + b_ref[...] + rt_ref[...], 0.0)
        ids = bt_ref[...]
        oh = (lax.broadcasted_iota(jnp.int32, (NUM_GRAPHS, N), 0) == ids
              ).astype(jnp.float32)
        pooled = jnp.dot(oh, h, preferred_element_type=jnp.float32)
        t = jnp.maximum(
            jnp.dot(pooled, w1_ref[...].T, preferred_element_type=jnp.float32)
            + c1_ref[...], 0.0)
        logits = (jnp.dot(t, w2_ref[...].T, preferred_element_type=jnp.float32)
                  + c2_ref[...])
        m = jnp.max(logits, axis=-1, keepdims=True)
        sh = logits - m
        o_ref[...] = sh - jnp.log(jnp.sum(jnp.exp(sh), axis=-1, keepdims=True))
    return pl.pallas_call(
        body,
        out_shape=jax.ShapeDtypeStruct((NUM_GRAPHS, NUM_CLASSES), jnp.float32),
    )(aggp, b5, root5, batch2d, Wl1, bl1, Wl2, bl2)


def kernel(x, edge_index, batch, edge_weight,
           W1_rel, b1, W1_root, W2_rel, b2, W2_root, W3_rel, b3, W3_root,
           W4_rel, b4, W4_root, W5_rel, b5, W5_root, Wl1, bl1, Wl2, bl2):
    srcm = edge_index[0].reshape(NW, NCHK, CH)
    dstm = edge_index[1].reshape(NW, NCHK, CH)
    wm = edge_weight.reshape(NW, 1, EPW)
    zer = jnp.zeros((NS, RPT, DIM), jnp.float32)

    r, root = _tc_first(x, jnp.concatenate([W1_rel, W1_root], axis=0))
    for b_prev, Wr, Wt in ((b1, W2_rel, W2_root), (b2, W3_rel, W3_root),
                           (b3, W4_rel, W4_root), (b4, W5_rel, W5_root)):
        aggp = _sc_aggregate(r, srcm, dstm, wm, zer).reshape(NC, N, DIM)
        r, root = _tc_mid(aggp, b_prev.reshape(1, DIM), root,
                          jnp.concatenate([Wr, Wt], axis=0))
    aggp = _sc_aggregate(r, srcm, dstm, wm, zer).reshape(NC, N, DIM)
    return _tc_final(aggp, b5.reshape(1, DIM), root, batch.reshape(1, N),
                     Wl1, bl1.reshape(1, DIM), Wl2, bl2.reshape(1, NUM_CLASSES))


# speedup vs baseline: 2.2306x; 1.0120x over previous
import functools

import jax
import jax.numpy as jnp
from jax import lax
from jax.experimental import pallas as pl
from jax.experimental.pallas import tpu as pltpu
from jax.experimental.pallas import tpu_sc as plsc

N = 10000
E = 320000
FEAT = 128
DIM = 32
NUM_GRAPHS = 64
NUM_CLASSES = 10

NC = 2
NS = 16
NW = NC * NS
CH = 80
SPB = 5
BLK = CH * SPB
NBLK = 25
EPW = BLK * NBLK
EP = NW * EPW
NCHK = EPW // CH
GPB = BLK // 16
RPT = N // NS


def _sc_aggregate(r, srcm, dstm, wm, zer):
    mesh = plsc.VectorSubcoreMesh(
        core_axis_name="c", subcore_axis_name="s", num_cores=NC, num_subcores=NS)

    @functools.partial(
        pl.kernel,
        out_type=jax.ShapeDtypeStruct((NC, NS, RPT, DIM), jnp.float32),
        mesh=mesh,
        scratch_types=[
            pltpu.VMEM((NCHK, CH), jnp.int32),
            pltpu.VMEM((NCHK, CH), jnp.int32),
            pltpu.VMEM((EPW + 16,), jnp.float32),
            pltpu.VMEM((BLK, DIM), jnp.float32),
            pltpu.VMEM((BLK, DIM), jnp.float32),
            pltpu.VMEM((BLK, DIM), jnp.float32),
            pltpu.VMEM((BLK, DIM), jnp.float32),
            pltpu.VMEM_SHARED((N, DIM), jnp.float32),
            pltpu.SemaphoreType.DMA,
            pltpu.SemaphoreType.DMA,
            pltpu.SemaphoreType.DMA,
        ],
        compiler_params=pltpu.CompilerParams(use_tc_tiling_on_sc=False),
    )
    def agg(r_hbm, src_hbm, dst_hbm, w_hbm, z_hbm, out_hbm,
            src_v, dst_v, w_v, rows_a, rows_b, rows_c, rows_d,
            acc_sh, gsem, ssem, zsem):
        c = lax.axis_index("c")
        s = lax.axis_index("s")
        wid = s * NC + c
        rows = (rows_a, rows_b, rows_c, rows_d)

        zd = pltpu.async_copy(z_hbm.at[s], acc_sh.at[pl.ds(s * RPT, RPT)], zsem)

        pltpu.sync_copy(src_hbm.at[wid], src_v)
        pltpu.sync_copy(dst_hbm.at[wid], dst_v)
        pltpu.sync_copy(w_hbm.at[wid, 0], w_v.at[pl.ds(0, EPW)])

        def fire_gathers(b):
            buf = rows[b % 4]
            return [pltpu.async_copy(r_hbm.at[src_v.at[b * SPB + j]],
                                     buf.at[pl.ds(j * CH, CH)], gsem)
                    for j in range(SPB)]

        def fire_scatters(b):
            buf = rows[b % 4]
            return [pltpu.async_copy(buf.at[pl.ds(j * CH, CH)],
                                     acc_sh.at[dst_v.at[b * SPB + j]],
                                     ssem, add=True)
                    for j in range(SPB)]

        gds = [None] * NBLK
        sds = [None] * NBLK
        gds[0] = fire_gathers(0)
        gds[1] = fire_gathers(1)
        for b in range(NBLK):
            buf = rows[b % 4]
            if b + 2 < NBLK:
                if b >= 2:
                    for d_ in sds[b - 2]:
                        d_.wait()
                gds[b + 2] = fire_gathers(b + 2)
            for d_ in gds[b]:
                d_.wait()
            if b == 0:
                zd.wait()
                plsc.subcore_barrier()

            @plsc.parallel_loop(0, BLK, unroll=8)
            def _(e, b=b, buf=buf):
                wrow = w_v[pl.ds(b * BLK + e, 16)]
                wv = jnp.broadcast_to(wrow[0], (16,))
                buf[e, pl.ds(0, 16)] = buf[e, pl.ds(0, 16)] * wv
                buf[e, pl.ds(16, 16)] = buf[e, pl.ds(16, 16)] * wv
            sds[b] = fire_scatters(b)

        for b in (NBLK - 4, NBLK - 3, NBLK - 2, NBLK - 1):
            for d_ in sds[b]:
                d_.wait()
        plsc.subcore_barrier()
        pltpu.sync_copy(acc_sh.at[pl.ds(s * RPT, RPT)], out_hbm.at[c, s])

    return agg(r, srcm, dstm, wm, zer)


def _tc_first(x, Wc):
    def body(x_ref, w_ref, r_ref, root_ref):
        y = jnp.dot(x_ref[...], w_ref[...].T, preferred_element_type=jnp.float32)
        r_ref[...] = y[:, :DIM]
        root_ref[...] = y[:, DIM:]
    return pl.pallas_call(
        body,
        out_shape=(jax.ShapeDtypeStruct((N, DIM), jnp.float32),
                   jax.ShapeDtypeStruct((N, DIM), jnp.float32)),
    )(x, Wc)


def _tc_mid(aggp, b, root, Wc):
    def body(a_ref, b_ref, rt_ref, w_ref, r_ref, root_ref):
        h = jnp.maximum(a_ref[0] + a_ref[1] + b_ref[...] + rt_ref[...], 0.0)
        y = jnp.dot(h, w_ref[...].T, preferred_element_type=jnp.float32)
        r_ref[...] = y[:, :DIM]
        root_ref[...] = y[:, DIM:]
    return pl.pallas_call(
        body,
        out_shape=(jax.ShapeDtypeStruct((N, DIM), jnp.float32),
                   jax.ShapeDtypeStruct((N, DIM), jnp.float32)),
    )(aggp, b, root, Wc)


def _tc_final(aggp, b5, root5, batch2d, Wl1, bl1, Wl2, bl2):
    def body(a_ref, b_ref, rt_ref, bt_ref, w1_ref, c1_ref, w2_ref, c2_ref, o_ref):
        h = jnp.maximum(a_ref[0] + a_ref[1] + b_ref[...] + rt_ref[...], 0.0)
        ids = bt_ref[...]
        oh = (lax.broadcasted_iota(jnp.int32, (NUM_GRAPHS, N), 0) == ids
              ).astype(jnp.float32)
        pooled = jnp.dot(oh, h, preferred_element_type=jnp.float32)
        t = jnp.maximum(
            jnp.dot(pooled, w1_ref[...].T, preferred_element_type=jnp.float32)
            + c1_ref[...], 0.0)
        logits = (jnp.dot(t, w2_ref[...].T, preferred_element_type=jnp.float32)
                  + c2_ref[...])
        m = jnp.max(logits, axis=-1, keepdims=True)
        sh = logits - m
        o_ref[...] = sh - jnp.log(jnp.sum(jnp.exp(sh), axis=-1, keepdims=True))
    return pl.pallas_call(
        body,
        out_shape=jax.ShapeDtypeStruct((NUM_GRAPHS, NUM_CLASSES), jnp.float32),
    )(aggp, b5, root5, batch2d, Wl1, bl1, Wl2, bl2)


def kernel(x, edge_index, batch, edge_weight,
           W1_rel, b1, W1_root, W2_rel, b2, W2_root, W3_rel, b3, W3_root,
           W4_rel, b4, W4_root, W5_rel, b5, W5_root, Wl1, bl1, Wl2, bl2):
    srcm = edge_index[0].reshape(NW, NCHK, CH)
    dstm = edge_index[1].reshape(NW, NCHK, CH)
    wm = edge_weight.reshape(NW, 1, EPW)
    zer = jnp.zeros((NS, RPT, DIM), jnp.float32)

    r, root = _tc_first(x, jnp.concatenate([W1_rel, W1_root], axis=0))
    for b_prev, Wr, Wt in ((b1, W2_rel, W2_root), (b2, W3_rel, W3_root),
                           (b3, W4_rel, W4_root), (b4, W5_rel, W5_root)):
        aggp = _sc_aggregate(r, srcm, dstm, wm, zer).reshape(NC, N, DIM)
        r, root = _tc_mid(aggp, b_prev.reshape(1, DIM), root,
                          jnp.concatenate([Wr, Wt], axis=0))
    aggp = _sc_aggregate(r, srcm, dstm, wm, zer).reshape(NC, N, DIM)
    return _tc_final(aggp, b5.reshape(1, DIM), root, batch.reshape(1, N),
                     Wl1, bl1.reshape(1, DIM), Wl2, bl2.reshape(1, NUM_CLASSES))


# speedup vs baseline: 2.2611x; 1.0137x over previous
import functools

import jax
import jax.numpy as jnp
from jax import lax
from jax.experimental import pallas as pl
from jax.experimental.pallas import tpu as pltpu
from jax.experimental.pallas import tpu_sc as plsc

N = 10000
E = 320000
FEAT = 128
DIM = 32
NUM_GRAPHS = 64
NUM_CLASSES = 10

NC = 2
NS = 16
NW = NC * NS
CH = 80
SPB = 5
BLK = CH * SPB
NBLK = 25
EPW = BLK * NBLK
EP = NW * EPW
NCHK = EPW // CH
GPB = BLK // 16
RPT = N // NS


def _sc_aggregate(r, srcm, dstm, wm, zer):
    mesh = plsc.VectorSubcoreMesh(
        core_axis_name="c", subcore_axis_name="s", num_cores=NC, num_subcores=NS)

    @functools.partial(
        pl.kernel,
        out_type=jax.ShapeDtypeStruct((NC, NS, RPT, DIM), jnp.float32),
        mesh=mesh,
        scratch_types=[
            pltpu.VMEM((NCHK, CH), jnp.int32),
            pltpu.VMEM((NCHK, CH), jnp.int32),
            pltpu.VMEM((EPW + 16,), jnp.float32),
            pltpu.VMEM((BLK, DIM), jnp.float32),
            pltpu.VMEM((BLK, DIM), jnp.float32),
            pltpu.VMEM((BLK, DIM), jnp.float32),
            pltpu.VMEM((BLK, DIM), jnp.float32),
            pltpu.VMEM_SHARED((N, DIM), jnp.float32),
            pltpu.SemaphoreType.DMA,
            pltpu.SemaphoreType.DMA,
            pltpu.SemaphoreType.DMA,
        ],
        compiler_params=pltpu.CompilerParams(use_tc_tiling_on_sc=False),
    )
    def agg(r_hbm, src_hbm, dst_hbm, w_hbm, z_hbm, out_hbm,
            src_v, dst_v, w_v, rows_a, rows_b, rows_c, rows_d,
            acc_sh, gsem, ssem, zsem):
        c = lax.axis_index("c")
        s = lax.axis_index("s")
        wid = s * NC + c
        rows = (rows_a, rows_b, rows_c, rows_d)

        zd = pltpu.async_copy(z_hbm.at[s], acc_sh.at[pl.ds(s * RPT, RPT)], zsem)

        pltpu.sync_copy(src_hbm.at[wid], src_v)
        dd = pltpu.async_copy(dst_hbm.at[wid], dst_v, zsem)
        wd = pltpu.async_copy(w_hbm.at[wid, 0], w_v.at[pl.ds(0, EPW)], zsem)

        def fire_gathers(b):
            buf = rows[b % 4]
            return [pltpu.async_copy(r_hbm.at[src_v.at[b * SPB + j]],
                                     buf.at[pl.ds(j * CH, CH)], gsem)
                    for j in range(SPB)]

        def fire_scatters(b):
            buf = rows[b % 4]
            return [pltpu.async_copy(buf.at[pl.ds(j * CH, CH)],
                                     acc_sh.at[dst_v.at[b * SPB + j]],
                                     ssem, add=True)
                    for j in range(SPB)]

        gds = [None] * NBLK
        sds = [None] * NBLK
        gds[0] = fire_gathers(0)
        gds[1] = fire_gathers(1)
        for b in range(NBLK):
            buf = rows[b % 4]
            if b + 2 < NBLK:
                if b >= 2:
                    for d_ in sds[b - 2]:
                        d_.wait()
                gds[b + 2] = fire_gathers(b + 2)
            for d_ in gds[b]:
                d_.wait()
            if b == 0:
                dd.wait()
                wd.wait()
                zd.wait()
                plsc.subcore_barrier()

            @plsc.parallel_loop(0, BLK, unroll=8)
            def _(e, b=b, buf=buf):
                wrow = w_v[pl.ds(b * BLK + e, 16)]
                wv = jnp.broadcast_to(wrow[0], (16,))
                buf[e, pl.ds(0, 16)] = buf[e, pl.ds(0, 16)] * wv
                buf[e, pl.ds(16, 16)] = buf[e, pl.ds(16, 16)] * wv
            sds[b] = fire_scatters(b)

        for b in (NBLK - 4, NBLK - 3, NBLK - 2, NBLK - 1):
            for d_ in sds[b]:
                d_.wait()
        plsc.subcore_barrier()
        pltpu.sync_copy(acc_sh.at[pl.ds(s * RPT, RPT)], out_hbm.at[c, s])

    return agg(r, srcm, dstm, wm, zer)


def _tc_first(x, Wc):
    def body(x_ref, w_ref, r_ref, root_ref):
        y = jnp.dot(x_ref[...], w_ref[...].T, preferred_element_type=jnp.float32)
        r_ref[...] = y[:, :DIM]
        root_ref[...] = y[:, DIM:]
    return pl.pallas_call(
        body,
        out_shape=(jax.ShapeDtypeStruct((N, DIM), jnp.float32),
                   jax.ShapeDtypeStruct((N, DIM), jnp.float32)),
    )(x, Wc)


def _tc_mid(aggp, b, root, Wc):
    def body(a_ref, b_ref, rt_ref, w_ref, r_ref, root_ref):
        h = jnp.maximum(a_ref[0] + a_ref[1] + b_ref[...] + rt_ref[...], 0.0)
        y = jnp.dot(h, w_ref[...].T, preferred_element_type=jnp.float32)
        r_ref[...] = y[:, :DIM]
        root_ref[...] = y[:, DIM:]
    return pl.pallas_call(
        body,
        out_shape=(jax.ShapeDtypeStruct((N, DIM), jnp.float32),
                   jax.ShapeDtypeStruct((N, DIM), jnp.float32)),
    )(aggp, b, root, Wc)


def _tc_final(aggp, b5, root5, batch2d, Wl1, bl1, Wl2, bl2):
    def body(a_ref, b_ref, rt_ref, bt_ref, w1_ref, c1_ref, w2_ref, c2_ref, o_ref):
        h = jnp.maximum(a_ref[0] + a_ref[1] + b_ref[...] + rt_ref[...], 0.0)
        ids = bt_ref[...]
        oh = (lax.broadcasted_iota(jnp.int32, (NUM_GRAPHS, N), 0) == ids
              ).astype(jnp.float32)
        pooled = jnp.dot(oh, h, preferred_element_type=jnp.float32)
        t = jnp.maximum(
            jnp.dot(pooled, w1_ref[...].T, preferred_element_type=jnp.float32)
            + c1_ref[...], 0.0)
        logits = (jnp.dot(t, w2_ref[...].T, preferred_element_type=jnp.float32)
                  + c2_ref[...])
        m = jnp.max(logits, axis=-1, keepdims=True)
        sh = logits - m
        o_ref[...] = sh - jnp.log(jnp.sum(jnp.exp(sh), axis=-1, keepdims=True))
    return pl.pallas_call(
        body,
        out_shape=jax.ShapeDtypeStruct((NUM_GRAPHS, NUM_CLASSES), jnp.float32),
    )(aggp, b5, root5, batch2d, Wl1, bl1, Wl2, bl2)


def kernel(x, edge_index, batch, edge_weight,
           W1_rel, b1, W1_root, W2_rel, b2, W2_root, W3_rel, b3, W3_root,
           W4_rel, b4, W4_root, W5_rel, b5, W5_root, Wl1, bl1, Wl2, bl2):
    srcm = edge_index[0].reshape(NW, NCHK, CH)
    dstm = edge_index[1].reshape(NW, NCHK, CH)
    wm = edge_weight.reshape(NW, 1, EPW)
    zer = jnp.zeros((NS, RPT, DIM), jnp.float32)

    r, root = _tc_first(x, jnp.concatenate([W1_rel, W1_root], axis=0))
    for b_prev, Wr, Wt in ((b1, W2_rel, W2_root), (b2, W3_rel, W3_root),
                           (b3, W4_rel, W4_root), (b4, W5_rel, W5_root)):
        aggp = _sc_aggregate(r, srcm, dstm, wm, zer).reshape(NC, N, DIM)
        r, root = _tc_mid(aggp, b_prev.reshape(1, DIM), root,
                          jnp.concatenate([Wr, Wt], axis=0))
    aggp = _sc_aggregate(r, srcm, dstm, wm, zer).reshape(NC, N, DIM)
    return _tc_final(aggp, b5.reshape(1, DIM), root, batch.reshape(1, N),
                     Wl1, bl1.reshape(1, DIM), Wl2, bl2.reshape(1, NUM_CLASSES))


# speedup vs baseline: 2.2634x; 1.0010x over previous
import functools

import jax
import jax.numpy as jnp
from jax import lax
from jax.experimental import pallas as pl
from jax.experimental.pallas import tpu as pltpu
from jax.experimental.pallas import tpu_sc as plsc

N = 10000
E = 320000
FEAT = 128
DIM = 32
NUM_GRAPHS = 64
NUM_CLASSES = 10

NC = 2
NS = 16
NW = NC * NS
CH = 80
SPB = 5
BLK = CH * SPB
NBLK = 25
EPW = BLK * NBLK
NCHK = EPW // CH
RPT = N // NS


def _sc_aggregate(r, srcm, dstm, wm, zer):
    mesh = plsc.VectorSubcoreMesh(
        core_axis_name="c", subcore_axis_name="s", num_cores=NC, num_subcores=NS)

    @functools.partial(
        pl.kernel,
        out_type=jax.ShapeDtypeStruct((NC, NS, RPT, DIM), jnp.float32),
        mesh=mesh,
        scratch_types=[
            pltpu.VMEM((NCHK, CH), jnp.int32),
            pltpu.VMEM((NCHK, CH), jnp.int32),
            pltpu.VMEM((EPW + 16,), jnp.float32),
            pltpu.VMEM((BLK, DIM), jnp.float32),
            pltpu.VMEM((BLK, DIM), jnp.float32),
            pltpu.VMEM((BLK, DIM), jnp.float32),
            pltpu.VMEM((BLK, DIM), jnp.float32),
            pltpu.VMEM_SHARED((N, DIM), jnp.float32),
            pltpu.SemaphoreType.DMA,
            pltpu.SemaphoreType.DMA,
            pltpu.SemaphoreType.DMA,
        ],
        compiler_params=pltpu.CompilerParams(use_tc_tiling_on_sc=False),
    )
    def agg(r_hbm, src_hbm, dst_hbm, w_hbm, z_hbm, out_hbm,
            src_v, dst_v, w_v, rows_a, rows_b, rows_c, rows_d,
            acc_sh, gsem, ssem, zsem):
        c = lax.axis_index("c")
        s = lax.axis_index("s")
        wid = s * NC + c
        rows = (rows_a, rows_b, rows_c, rows_d)

        zd = pltpu.async_copy(z_hbm.at[s], acc_sh.at[pl.ds(s * RPT, RPT)], zsem)

        pltpu.sync_copy(src_hbm.at[wid], src_v)
        dd = pltpu.async_copy(dst_hbm.at[wid], dst_v, zsem)
        wd = pltpu.async_copy(w_hbm.at[wid, 0], w_v.at[pl.ds(0, EPW)], zsem)

        def fire_gathers(b):
            buf = rows[b % 4]
            return [pltpu.async_copy(r_hbm.at[src_v.at[b * SPB + j]],
                                     buf.at[pl.ds(j * CH, CH)], gsem)
                    for j in range(SPB)]

        def fire_scatters(b):
            buf = rows[b % 4]
            return [pltpu.async_copy(buf.at[pl.ds(j * CH, CH)],
                                     acc_sh.at[dst_v.at[b * SPB + j]],
                                     ssem, add=True)
                    for j in range(SPB)]

        gds = [None] * NBLK
        sds = [None] * NBLK
        gds[0] = fire_gathers(0)
        gds[1] = fire_gathers(1)
        for b in range(NBLK):
            buf = rows[b % 4]
            if b + 2 < NBLK:
                if b >= 2:
                    for d_ in sds[b - 2]:
                        d_.wait()
                gds[b + 2] = fire_gathers(b + 2)
            for d_ in gds[b]:
                d_.wait()
            if b == 0:
                dd.wait()
                wd.wait()
                zd.wait()
                plsc.subcore_barrier()

            @plsc.parallel_loop(0, BLK, unroll=8)
            def _(e, b=b, buf=buf):
                wrow = w_v[pl.ds(b * BLK + e, 16)]
                wv = jnp.broadcast_to(wrow[0], (16,))
                buf[e, pl.ds(0, 16)] = buf[e, pl.ds(0, 16)] * wv
                buf[e, pl.ds(16, 16)] = buf[e, pl.ds(16, 16)] * wv
            sds[b] = fire_scatters(b)

        for b in (NBLK - 4, NBLK - 3, NBLK - 2, NBLK - 1):
            for d_ in sds[b]:
                d_.wait()
        plsc.subcore_barrier()
        pltpu.sync_copy(acc_sh.at[pl.ds(s * RPT, RPT)], out_hbm.at[c, s])

    return agg(r, srcm, dstm, wm, zer)


def _tc_first(x, Wc):
    def body(x_ref, w_ref, r_ref, root_ref):
        y = jnp.dot(x_ref[...], w_ref[...].T, preferred_element_type=jnp.float32)
        r_ref[...] = y[:, :DIM]
        root_ref[...] = y[:, DIM:]
    return pl.pallas_call(
        body,
        out_shape=(jax.ShapeDtypeStruct((N, DIM), jnp.float32),
                   jax.ShapeDtypeStruct((N, DIM), jnp.float32)),
    )(x, Wc)


def _tc_mid(aggp, b, root, Wc):
    def body(a_ref, b_ref, rt_ref, w_ref, r_ref, root_ref):
        h = jnp.maximum(a_ref[0] + a_ref[1] + b_ref[...] + rt_ref[...], 0.0)
        y = jnp.dot(h, w_ref[...].T, preferred_element_type=jnp.float32)
        r_ref[...] = y[:, :DIM]
        root_ref[...] = y[:, DIM:]
    return pl.pallas_call(
        body,
        out_shape=(jax.ShapeDtypeStruct((N, DIM), jnp.float32),
                   jax.ShapeDtypeStruct((N, DIM), jnp.float32)),
    )(aggp, b, root, Wc)


def _tc_final(aggp, b5, root5, batch2d, Wl1, bl1, Wl2, bl2):
    def body(a_ref, b_ref, rt_ref, bt_ref, w1_ref, c1_ref, w2_ref, c2_ref, o_ref):
        h = jnp.maximum(a_ref[0] + a_ref[1] + b_ref[...] + rt_ref[...], 0.0)
        ids = bt_ref[...]
        oh = (lax.broadcasted_iota(jnp.int32, (NUM_GRAPHS, N), 0) == ids
              ).astype(jnp.float32)
        pooled = jnp.dot(oh, h, preferred_element_type=jnp.float32)
        t = jnp.maximum(
            jnp.dot(pooled, w1_ref[...].T, preferred_element_type=jnp.float32)
            + c1_ref[...], 0.0)
        logits = (jnp.dot(t, w2_ref[...].T, preferred_element_type=jnp.float32)
                  + c2_ref[...])
        m = jnp.max(logits, axis=-1, keepdims=True)
        sh = logits - m
        o_ref[...] = sh - jnp.log(jnp.sum(jnp.exp(sh), axis=-1, keepdims=True))
    return pl.pallas_call(
        body,
        out_shape=jax.ShapeDtypeStruct((NUM_GRAPHS, NUM_CLASSES), jnp.float32),
    )(aggp, b5, root5, batch2d, Wl1, bl1, Wl2, bl2)


def kernel(x, edge_index, batch, edge_weight,
           W1_rel, b1, W1_root, W2_rel, b2, W2_root, W3_rel, b3, W3_root,
           W4_rel, b4, W4_root, W5_rel, b5, W5_root, Wl1, bl1, Wl2, bl2):
    srcm = edge_index[0].reshape(NW, NCHK, CH)
    dstm = edge_index[1].reshape(NW, NCHK, CH)
    wm = edge_weight.reshape(NW, 1, EPW)
    zer = jnp.zeros((NS, RPT, DIM), jnp.float32)

    r, root = _tc_first(x, jnp.concatenate([W1_rel, W1_root], axis=0))
    for b_prev, Wr, Wt in ((b1, W2_rel, W2_root), (b2, W3_rel, W3_root),
                           (b3, W4_rel, W4_root), (b4, W5_rel, W5_root)):
        aggp = _sc_aggregate(r, srcm, dstm, wm, zer).reshape(NC, N, DIM)
        r, root = _tc_mid(aggp, b_prev.reshape(1, DIM), root,
                          jnp.concatenate([Wr, Wt], axis=0))
    aggp = _sc_aggregate(r, srcm, dstm, wm, zer).reshape(NC, N, DIM)
    return _tc_final(aggp, b5.reshape(1, DIM), root, batch.reshape(1, N),
                     Wl1, bl1.reshape(1, DIM), Wl2, bl2.reshape(1, NUM_CLASSES))
